# baseline (speedup 1.0000x reference)
import jax
import jax.numpy as jnp
from jax.experimental import pallas as pl
from jax.experimental.pallas import tpu as pltpu

CONF = 0.5
GROUP = 5
SUB = 8
FRAMES = 8


def _body(det_ref, sv_ref, o_ref):
    x = det_ref[:, 0]
    lanes = x.shape[-1]
    l = jax.lax.broadcasted_iota(jnp.int32, (SUB, lanes), 1)
    s = jax.lax.broadcasted_iota(jnp.int32, (SUB, lanes), 0)
    ii = s * lanes + l
    m = l % GROUP
    is_score = m == 0
    is_last = m == GROUP - 1

    fail = jnp.logical_and(is_score, x < CONF)
    cand = jnp.where(fail, ii, jnp.int32(2**30))
    first_fail = jnp.min(cand, axis=(1, 2), keepdims=True)
    valid = ii < first_fail

    boxes = pltpu.roll(x, lanes - 1, 2)
    score = pltpu.roll(x, 4, 2)
    val = jnp.where(is_last, score, boxes * sv_ref[:])
    o_ref[:] = jnp.where(valid, val, 0.0)


def kernel(detections, scale):
    B, C, K, G = detections.shape
    lanes = K * G // SUB
    det = detections.reshape(B, C, SUB, lanes)

    lane_idx = jnp.arange(lanes, dtype=jnp.int32)
    m = lane_idx % GROUP
    sv = jnp.where(m == GROUP - 1, 1.0,
                   scale[jnp.minimum(m, 3)]).reshape(1, lanes)

    out = pl.pallas_call(
        _body,
        grid=(B // FRAMES,),
        in_specs=[
            pl.BlockSpec((FRAMES, 1, SUB, lanes), lambda i: (i, 1, 0, 0)),
            pl.BlockSpec((1, lanes), lambda i: (0, 0)),
        ],
        out_specs=pl.BlockSpec((FRAMES, SUB, lanes), lambda i: (i, 0, 0)),
        out_shape=jax.ShapeDtypeStruct((B, SUB, lanes), jnp.float32),
        compiler_params=pltpu.CompilerParams(
            dimension_semantics=("parallel",)),
    )(det, sv)
    return out.reshape(B, K, G)


# speedup vs baseline: 3.7597x; 3.7597x over previous
import jax
import jax.numpy as jnp
from jax.experimental import pallas as pl
from jax.experimental.pallas import tpu as pltpu

CONF = 0.5
FRAMES = 8
LS = 128


def _body(det_ref, sv_ref, o_ref):
    score = det_ref[:, 1:LS:2, :]
    t = jax.lax.broadcasted_iota(jnp.int32, (64, LS), 0)
    kl = jax.lax.broadcasted_iota(jnp.int32, (64, LS), 1)
    ii = t * LS + kl
    fail = score < CONF
    cand = jnp.where(fail, ii, jnp.int32(2**30))
    first_fail = jnp.min(cand, axis=(1, 2), keepdims=True)
    valid = ii < first_fail

    for j in range(4):
        base = (j + 1) * LS
        box = det_ref[:, base + 1:base + LS:2, :]
        o_ref[j] = jnp.where(valid, box * sv_ref[j], 0.0)
    o_ref[4] = jnp.where(valid, score, 0.0)


def kernel(detections, scale):
    B, C, K, G = detections.shape
    nt = K // LS
    det = (detections.transpose(0, 3, 1, 2)
           .reshape(B, G, C, nt, LS)
           .transpose(0, 1, 3, 2, 4)
           .reshape(B, G * C * LS // 2, LS))

    sv = jnp.broadcast_to(scale[:, None], (4, LS))

    out = pl.pallas_call(
        _body,
        grid=(B // FRAMES,),
        in_specs=[
            pl.BlockSpec((FRAMES, G * C * LS // 2, LS), lambda i: (i, 0, 0)),
            pl.BlockSpec((4, LS), lambda i: (0, 0)),
        ],
        out_specs=pl.BlockSpec((G, FRAMES, nt, LS), lambda i: (0, i, 0, 0)),
        out_shape=jax.ShapeDtypeStruct((G, B, nt, LS), jnp.float32),
        compiler_params=pltpu.CompilerParams(
            dimension_semantics=("parallel",)),
    )(det, sv)
    return out.transpose(1, 2, 3, 0).reshape(B, K, G)
